# baseline (speedup 1.0000x reference)
import functools

import jax
import jax.numpy as jnp
from jax import lax
from jax.experimental import pallas as pl
from jax.experimental.pallas import tpu as pltpu
from jax.experimental.pallas import tpu_sc as plsc

VOCAB = 1000000
EMBED = 64
BATCH = 16384
NEG = 5
NIDX = NEG + 1
WIDE = 128

NC, NS, LANES = 2, 16, 16
NW = NC * NS
BPW = BATCH // NW
CB = 128
NCHUNK = BPW // CB
NGROUP = CB // LANES
SROWS = 8


def _sc_scores(pos_w, vidx, w_table, v_table):
    mesh = plsc.VectorSubcoreMesh(core_axis_name="c", subcore_axis_name="s")

    @functools.partial(
        pl.kernel,
        out_type=jax.ShapeDtypeStruct((NW, NCHUNK, SROWS, CB), jnp.float32),
        mesh=mesh,
        compiler_params=pltpu.CompilerParams(needs_layout_passes=False),
        scratch_types=[
            pltpu.VMEM((CB,), jnp.int32),
            pltpu.VMEM((NIDX, CB), jnp.int32),
            pltpu.VMEM((CB,), jnp.int32),
            pltpu.VMEM((NIDX, CB), jnp.int32),
            pltpu.VMEM((CB, WIDE), jnp.float32),
            pltpu.VMEM((NIDX, CB, WIDE), jnp.float32),
            pltpu.VMEM((SROWS, CB), jnp.float32),
            pltpu.SemaphoreType.DMA,
        ],
    )
    def k(pos_w_hbm, vidx_hbm, w_hbm, v_hbm, out_hbm,
          widx_v, vidx_v, widx2_v, vidx2_v, wrows, vrows, scores_v, sem):
        wid = lax.axis_index("s") * NC + lax.axis_index("c")
        lane = lax.iota(jnp.int32, LANES)

        zero = jnp.zeros((LANES,), jnp.float32)
        for r in range(NIDX, SROWS):
            for g in range(NGROUP):
                scores_v[r, pl.ds(g * LANES, LANES)] = zero

        for chunk in range(NCHUNK):
            base = wid * BPW + chunk * CB
            pltpu.sync_copy(pos_w_hbm.at[pl.ds(base, CB)], widx_v)
            for j in range(NIDX):
                pltpu.sync_copy(vidx_hbm.at[j, pl.ds(base, CB)], vidx_v.at[j])
            for g in range(NGROUP):
                sl = pl.ds(g * LANES, LANES)
                widx2_v[sl] = widx_v[sl] >> 1
                for j in range(NIDX):
                    vidx2_v[j, sl] = vidx_v[j, sl] >> 1
            cps = [pltpu.async_copy(w_hbm.at[widx2_v], wrows, sem)]
            for j in range(NIDX):
                cps.append(pltpu.async_copy(v_hbm.at[vidx2_v.at[j]],
                                            vrows.at[j], sem))
            for cp in cps:
                cp.wait()

            for g in range(NGROUP):
                sl = pl.ds(g * LANES, LANES)
                i_vec = jnp.full((LANES,), g * LANES, jnp.int32) + lane
                wo = (widx_v[sl] & 1) << 6
                vo = [(vidx_v[j, sl] & 1) << 6 for j in range(NIDX)]

                def body(d, accs, i_vec=i_vec, wo=wo, vo=vo):
                    d_vec = jnp.full((LANES,), d, jnp.int32)
                    wv = plsc.load_gather(wrows, [i_vec, wo + d_vec])
                    return tuple(
                        accs[j] + wv * plsc.load_gather(
                            vrows,
                            [jnp.full((LANES,), j, jnp.int32), i_vec,
                             vo[j] + d_vec])
                        for j in range(NIDX))

                accs = lax.fori_loop(0, EMBED, body, (zero,) * NIDX)
                scores_v[0, sl] = accs[0]
                for j in range(1, NIDX):
                    scores_v[j, sl] = -accs[j]

            pltpu.sync_copy(scores_v, out_hbm.at[wid, chunk])

    return k(pos_w, vidx, w_table, v_table)


def _tc_loss_body(x_ref, o_ref):
    x = jnp.clip(x_ref[...], -10.0, 10.0)
    row = lax.broadcasted_iota(jnp.int32, x.shape, 0)
    valid = (row % SROWS) < NIDX
    o_ref[0, 0] = -jnp.sum(jnp.where(valid, jax.nn.log_sigmoid(x), 0.0))


def kernel(pos_w, pos_v, neg_v, w_embeddings, v_embeddings):
    pos_w = jnp.asarray(pos_w, jnp.int32)
    vidx = jnp.concatenate(
        [jnp.asarray(pos_v, jnp.int32)[None, :],
         jnp.asarray(neg_v, jnp.int32).T], axis=0)
    w2 = w_embeddings.reshape(VOCAB // 2, WIDE)
    v2 = v_embeddings.reshape(VOCAB // 2, WIDE)

    scores = _sc_scores(pos_w, vidx, w2, v2)
    flat = scores.reshape(NW * NCHUNK * SROWS, CB)

    loss = pl.pallas_call(
        _tc_loss_body,
        out_shape=jax.ShapeDtypeStruct((1, 1), jnp.float32),
        out_specs=pl.BlockSpec(memory_space=pltpu.SMEM),
    )(flat)
    return loss[0, 0]


# speedup vs baseline: 1.4551x; 1.4551x over previous
import functools

import jax
import jax.numpy as jnp
from jax import lax
from jax.experimental import pallas as pl
from jax.experimental.pallas import tpu as pltpu
from jax.experimental.pallas import tpu_sc as plsc

VOCAB = 1000000
EMBED = 64
BATCH = 16384
NEG = 5
NIDX = NEG + 1
NROW = NIDX + 1

NC, NS, LANES = 2, 16, 16
NW = NC * NS
BPW = BATCH // NW
CB = 128
NCHUNK = BPW // CB
NGROUP = CB // LANES


def _sc_scores(pos_w, vidx, w_table, v_table):
    mesh = plsc.VectorSubcoreMesh(core_axis_name="c", subcore_axis_name="s")

    @functools.partial(
        pl.kernel,
        out_type=jax.ShapeDtypeStruct((NW, NCHUNK, NIDX, CB), jnp.float32),
        mesh=mesh,
        compiler_params=pltpu.CompilerParams(needs_layout_passes=False),
        scratch_types=[
            pltpu.VMEM((CB,), jnp.int32),
            pltpu.VMEM((NIDX, CB), jnp.int32),
            pltpu.VMEM((CB, EMBED), jnp.float32),
            pltpu.VMEM((NIDX, CB, EMBED), jnp.float32),
            pltpu.VMEM((NIDX, CB), jnp.float32),
            pltpu.SemaphoreType.DMA,
        ],
    )
    def k(pos_w_hbm, vidx_hbm, w_hbm, v_hbm, out_hbm,
          widx_v, vidx_v, wrows, vrows, scores_v, sem):
        wid = lax.axis_index("s") * NC + lax.axis_index("c")
        lane = lax.iota(jnp.int32, LANES)
        zero = jnp.zeros((LANES,), jnp.float32)

        for chunk in range(NCHUNK):
            base = wid * BPW + chunk * CB
            pltpu.sync_copy(pos_w_hbm.at[pl.ds(base, CB)], widx_v)
            for j in range(NIDX):
                pltpu.sync_copy(vidx_hbm.at[j, pl.ds(base, CB)], vidx_v.at[j])

            def enq(g, _):
                wvec = widx_v[pl.ds(g * LANES, LANES)]
                vvecs = [vidx_v[j, pl.ds(g * LANES, LANES)]
                         for j in range(NIDX)]
                for l in range(LANES):
                    i = g * LANES + l
                    pltpu.async_copy(w_hbm.at[wvec[l]], wrows.at[i], sem)
                    for j in range(NIDX):
                        pltpu.async_copy(v_hbm.at[vvecs[j][l]],
                                         vrows.at[j, i], sem)
                return 0

            lax.fori_loop(0, NGROUP, enq, 0)

            def drain(i, _):
                pltpu.make_async_copy(w_hbm.at[0], wrows.at[0], sem).wait()
                return 0

            lax.fori_loop(0, NROW * CB, drain, 0)

            for g in range(NGROUP):
                sl = pl.ds(g * LANES, LANES)
                i_vec = jnp.full((LANES,), g * LANES, jnp.int32) + lane

                def body(d, accs, i_vec=i_vec):
                    d_vec = jnp.full((LANES,), d, jnp.int32)
                    wv = plsc.load_gather(wrows, [i_vec, d_vec])
                    return tuple(
                        accs[j] + wv * plsc.load_gather(
                            vrows,
                            [jnp.full((LANES,), j, jnp.int32), i_vec, d_vec])
                        for j in range(NIDX))

                accs = lax.fori_loop(0, EMBED, body, (zero,) * NIDX)
                scores_v[0, sl] = accs[0]
                for j in range(1, NIDX):
                    scores_v[j, sl] = -accs[j]

            pltpu.sync_copy(scores_v, out_hbm.at[wid, chunk])

    return k(pos_w, vidx, w_table, v_table)


def _tc_loss_body(x_ref, o_ref):
    x = jnp.clip(x_ref[...], -10.0, 10.0)
    o_ref[0, 0] = -jnp.sum(jax.nn.log_sigmoid(x))


def kernel(pos_w, pos_v, neg_v, w_embeddings, v_embeddings):
    pos_w = jnp.asarray(pos_w, jnp.int32)
    vidx = jnp.concatenate(
        [jnp.asarray(pos_v, jnp.int32)[None, :],
         jnp.asarray(neg_v, jnp.int32).T], axis=0)

    scores = _sc_scores(pos_w, vidx, w_embeddings, v_embeddings)
    flat = scores.reshape(BATCH * NIDX // 128, 128)

    loss = pl.pallas_call(
        _tc_loss_body,
        out_shape=jax.ShapeDtypeStruct((1, 1), jnp.float32),
        out_specs=pl.BlockSpec(memory_space=pltpu.SMEM),
    )(flat)
    return loss[0, 0]


# speedup vs baseline: 1.4771x; 1.0151x over previous
import functools

import jax
import jax.numpy as jnp
from jax import lax
from jax.experimental import pallas as pl
from jax.experimental.pallas import tpu as pltpu
from jax.experimental.pallas import tpu_sc as plsc

VOCAB = 1000000
EMBED = 64
BATCH = 16384
NEG = 5
NIDX = NEG + 1
NROW = NIDX + 1

NC, NS, LANES = 2, 16, 16
NW = NC * NS
BPW = BATCH // NW
CB = 64
NCHUNK = BPW // CB
NGROUP = CB // LANES


def _sc_scores(pos_w, vidx, w_table, v_table):
    mesh = plsc.VectorSubcoreMesh(core_axis_name="c", subcore_axis_name="s")

    @functools.partial(
        pl.kernel,
        out_type=jax.ShapeDtypeStruct((NW, NCHUNK, NIDX, CB), jnp.float32),
        mesh=mesh,
        compiler_params=pltpu.CompilerParams(needs_layout_passes=False),
        scratch_types=[
            pltpu.VMEM((BPW,), jnp.int32),
            pltpu.VMEM((NIDX, BPW), jnp.int32),
            pltpu.VMEM((2, CB, EMBED), jnp.float32),
            pltpu.VMEM((2, NIDX, CB, EMBED), jnp.float32),
            pltpu.VMEM((NIDX, CB), jnp.float32),
            pltpu.SemaphoreType.DMA,
            pltpu.SemaphoreType.DMA,
        ],
    )
    def k(pos_w_hbm, vidx_hbm, w_hbm, v_hbm, out_hbm,
          widx_v, vidx_v, wrows, vrows, scores_v, sem0, sem1):
        wid = lax.axis_index("s") * NC + lax.axis_index("c")
        lane = lax.iota(jnp.int32, LANES)
        zero = jnp.zeros((LANES,), jnp.float32)
        sems = (sem0, sem1)
        base = wid * BPW

        icps = [pltpu.async_copy(pos_w_hbm.at[pl.ds(base, BPW)], widx_v,
                                 sem0)]
        for j in range(NIDX):
            icps.append(pltpu.async_copy(vidx_hbm.at[j, pl.ds(base, BPW)],
                                         vidx_v.at[j], sem0))
        for cp in icps:
            cp.wait()

        def enqueue(c, buf):
            sem = sems[buf]

            def enq(g, _):
                off = c * CB + g * LANES
                wvec = widx_v[pl.ds(off, LANES)]
                vvecs = [vidx_v[j, pl.ds(off, LANES)] for j in range(NIDX)]
                for l in range(LANES):
                    i = g * LANES + l
                    pltpu.async_copy(w_hbm.at[wvec[l]], wrows.at[buf, i],
                                     sem)
                    for j in range(NIDX):
                        pltpu.async_copy(v_hbm.at[vvecs[j][l]],
                                         vrows.at[buf, j, i], sem)
                return 0

            lax.fori_loop(0, NGROUP, enq, 0)

        def drain(buf):
            def one(i, _):
                pltpu.make_async_copy(w_hbm.at[0], wrows.at[buf, 0],
                                      sems[buf]).wait()
                return 0

            lax.fori_loop(0, NROW * CB, one, 0)

        def compute(c, buf):
            for g in range(NGROUP):
                sl = pl.ds(g * LANES, LANES)
                i_vec = jnp.full((LANES,), g * LANES, jnp.int32) + lane
                b_vec = jnp.full((LANES,), buf, jnp.int32)

                def body(d, accs, i_vec=i_vec, b_vec=b_vec):
                    d_vec = jnp.full((LANES,), d, jnp.int32)
                    wv = plsc.load_gather(wrows, [b_vec, i_vec, d_vec])
                    return tuple(
                        accs[j] + wv * plsc.load_gather(
                            vrows,
                            [b_vec, jnp.full((LANES,), j, jnp.int32), i_vec,
                             d_vec])
                        for j in range(NIDX))

                accs = lax.fori_loop(0, EMBED, body, (zero,) * NIDX)
                scores_v[0, sl] = accs[0]
                for j in range(1, NIDX):
                    scores_v[j, sl] = -accs[j]

            pltpu.sync_copy(scores_v, out_hbm.at[wid, c])

        enqueue(0, 0)
        for c in range(NCHUNK):
            if c + 1 < NCHUNK:
                enqueue(c + 1, (c + 1) % 2)
            drain(c % 2)
            compute(c, c % 2)

    return k(pos_w, vidx, w_table, v_table)


def _tc_loss_body(x_ref, o_ref):
    x = jnp.clip(x_ref[...], -10.0, 10.0)
    o_ref[0, 0] = -jnp.sum(jax.nn.log_sigmoid(x))


def kernel(pos_w, pos_v, neg_v, w_embeddings, v_embeddings):
    pos_w = jnp.asarray(pos_w, jnp.int32)
    vidx = jnp.concatenate(
        [jnp.asarray(pos_v, jnp.int32)[None, :],
         jnp.asarray(neg_v, jnp.int32).T], axis=0)

    scores = _sc_scores(pos_w, vidx, w_embeddings, v_embeddings)
    flat = scores.reshape(BATCH * NIDX // 128, 128)

    loss = pl.pallas_call(
        _tc_loss_body,
        out_shape=jax.ShapeDtypeStruct((1, 1), jnp.float32),
        out_specs=pl.BlockSpec(memory_space=pltpu.SMEM),
    )(flat)
    return loss[0, 0]
